# initial kernel scaffold (unmeasured)
import jax
import jax.numpy as jnp
from jax import lax
from jax.experimental import pallas as pl
from jax.experimental.pallas import tpu as pltpu

S_GLOBAL = 2048
S_HALF = 1024
N = 8192
K = 4096
NT = 16
TN = N // NT


def _matmul(A, W):

    def body(a_ref, w_ref, o_ref):
        o_ref[...] = jnp.dot(
            a_ref[...], w_ref[...], preferred_element_type=jnp.float32
        ).astype(jnp.bfloat16)

    return pl.pallas_call(
        body,
        grid=(NT,),
        in_specs=[
            pl.BlockSpec((S_GLOBAL, K), lambda j: (0, 0)),
            pl.BlockSpec((K, TN), lambda j: (0, j)),
        ],
        out_specs=pl.BlockSpec((S_GLOBAL, TN), lambda j: (0, j)),
        out_shape=jax.ShapeDtypeStruct((S_GLOBAL, N), jnp.bfloat16),
    )(A, W)


def _reduce_scatter_x(P):

    def body(p_ref, o_ref, recv_ref, send_sem, recv_sem):
        my_x = lax.axis_index("x")
        my_y = lax.axis_index("y")
        my_z = lax.axis_index("z")
        peer = (1 - my_x, my_y, my_z)

        barrier = pltpu.get_barrier_semaphore()
        pl.semaphore_signal(
            barrier, inc=1, device_id=peer, device_id_type=pl.DeviceIdType.MESH
        )
        pl.semaphore_wait(barrier, 1)

        rdma = pltpu.make_async_remote_copy(
            src_ref=p_ref.at[pl.ds((1 - my_x) * S_HALF, S_HALF), :],
            dst_ref=recv_ref,
            send_sem=send_sem,
            recv_sem=recv_sem,
            device_id=peer,
            device_id_type=pl.DeviceIdType.MESH,
        )
        rdma.start()
        rdma.wait()

        own = p_ref[pl.ds(my_x * S_HALF, S_HALF), :]
        o_ref[...] = own.astype(jnp.float32) + recv_ref[...].astype(jnp.float32)

    return pl.pallas_call(
        body,
        out_shape=jax.ShapeDtypeStruct((S_HALF, N), jnp.float32),
        in_specs=[pl.BlockSpec(memory_space=pltpu.VMEM)],
        out_specs=pl.BlockSpec(memory_space=pltpu.VMEM),
        scratch_shapes=[
            pltpu.VMEM((S_HALF, N), jnp.bfloat16),
            pltpu.SemaphoreType.DMA,
            pltpu.SemaphoreType.DMA,
        ],
        compiler_params=pltpu.CompilerParams(collective_id=0),
    )(P)


def kernel(O, Wo):
    B, S, H, D = O.shape
    A = O.reshape(S, H * D).astype(jnp.bfloat16)
    W = Wo.astype(jnp.bfloat16)
    P = _matmul(A, W)
    out = _reduce_scatter_x(P)
    return out.reshape(B, S_HALF, N)


# baseline (device time: 448410 ns/iter reference)
import jax
import jax.numpy as jnp
from jax import lax
from jax.experimental import pallas as pl
from jax.experimental.pallas import tpu as pltpu

S_GLOBAL = 2048
S_HALF = 1024
N = 8192
K = 4096
NT = 16
TN = N // NT


def _matmul(A, W):

    def body(a_ref, w_ref, o_ref):
        o_ref[...] = jnp.dot(
            a_ref[...], w_ref[...], preferred_element_type=jnp.float32
        ).astype(jnp.bfloat16)

    return pl.pallas_call(
        body,
        grid=(NT,),
        in_specs=[
            pl.BlockSpec((S_GLOBAL, K), lambda j: (0, 0)),
            pl.BlockSpec((K, TN), lambda j: (0, j)),
        ],
        out_specs=pl.BlockSpec((S_GLOBAL, TN), lambda j: (0, j)),
        out_shape=jax.ShapeDtypeStruct((S_GLOBAL, N), jnp.bfloat16),
    )(A, W)


def _exchange_x(P):

    def body(p_ref, o_ref, send_sem, recv_sem):
        my_x = lax.axis_index("x")
        my_y = lax.axis_index("y")
        my_z = lax.axis_index("z")
        peer = (1 - my_x, my_y, my_z)

        barrier = pltpu.get_barrier_semaphore()
        pl.semaphore_signal(
            barrier, inc=1, device_id=peer, device_id_type=pl.DeviceIdType.MESH
        )
        pl.semaphore_wait(barrier, 1)

        rdma = pltpu.make_async_remote_copy(
            src_ref=p_ref.at[pl.ds((1 - my_x) * S_HALF, S_HALF), :],
            dst_ref=o_ref,
            send_sem=send_sem,
            recv_sem=recv_sem,
            device_id=peer,
            device_id_type=pl.DeviceIdType.MESH,
        )
        rdma.start()
        rdma.wait()

    return pl.pallas_call(
        body,
        out_shape=jax.ShapeDtypeStruct((S_HALF, N), jnp.bfloat16),
        in_specs=[pl.BlockSpec(memory_space=pl.ANY)],
        out_specs=pl.BlockSpec(memory_space=pl.ANY),
        scratch_shapes=[
            pltpu.SemaphoreType.DMA,
            pltpu.SemaphoreType.DMA,
        ],
        compiler_params=pltpu.CompilerParams(collective_id=0),
    )(P)


def kernel(O, Wo):
    B, S, H, D = O.shape
    A = O.reshape(S, H * D).astype(jnp.bfloat16)
    W = Wo.astype(jnp.bfloat16)
    P = _matmul(A, W)
    recv = _exchange_x(P)
    my_x = lax.axis_index("x")
    own = lax.dynamic_slice(P, (my_x * S_HALF, 0), (S_HALF, N))
    out = own.astype(jnp.float32) + recv.astype(jnp.float32)
    return out.reshape(B, S_HALF, N)


# device time: 313071 ns/iter; 1.4323x vs baseline; 1.4323x over previous
import jax
import jax.numpy as jnp
from jax import lax
from jax.experimental import pallas as pl
from jax.experimental.pallas import tpu as pltpu

S_GLOBAL = 2048
S_HALF = 1024
N = 8192
K = 4096
NT = 32
TN = N // NT


def _fused(A2, W):

    def body(a_ref, w_ref, o_ref, send_ref, recv_ref, send_sems, recv_sems):
        m = pl.program_id(0)
        j = pl.program_id(1)

        my_x = lax.axis_index("x")
        my_y = lax.axis_index("y")
        my_z = lax.axis_index("z")
        peer = (1 - my_x, my_y, my_z)

        @pl.when((m == 0) & (j == 0))
        def _():
            barrier = pltpu.get_barrier_semaphore()
            pl.semaphore_signal(
                barrier, inc=1, device_id=peer,
                device_id_type=pl.DeviceIdType.MESH,
            )
            pl.semaphore_wait(barrier, 1)

        p = jnp.dot(a_ref[0], w_ref[...], preferred_element_type=jnp.float32)

        @pl.when(m == 0)
        def _():
            send_ref[pl.ds(j, 1), :, :] = p.astype(jnp.bfloat16)[None]
            rdma = pltpu.make_async_remote_copy(
                src_ref=send_ref.at[j],
                dst_ref=recv_ref.at[j],
                send_sem=send_sems.at[j],
                recv_sem=recv_sems.at[j],
                device_id=peer,
                device_id_type=pl.DeviceIdType.MESH,
            )
            rdma.start()

        @pl.when(m == 1)
        def _():
            rdma = pltpu.make_async_remote_copy(
                src_ref=send_ref.at[j],
                dst_ref=recv_ref.at[j],
                send_sem=send_sems.at[j],
                recv_sem=recv_sems.at[j],
                device_id=peer,
                device_id_type=pl.DeviceIdType.MESH,
            )
            rdma.wait()
            recv = recv_ref[pl.ds(j, 1), :, :][0]
            o_ref[...] = p + recv.astype(jnp.float32)

    return pl.pallas_call(
        body,
        grid=(2, NT),
        in_specs=[
            pl.BlockSpec((1, S_HALF, K), lambda m, j: (m, 0, 0)),
            pl.BlockSpec((K, TN), lambda m, j: (0, j)),
        ],
        out_specs=pl.BlockSpec((S_HALF, TN), lambda m, j: (0, m * j)),
        out_shape=jax.ShapeDtypeStruct((S_HALF, N), jnp.float32),
        scratch_shapes=[
            pltpu.VMEM((NT, S_HALF, TN), jnp.bfloat16),
            pltpu.VMEM((NT, S_HALF, TN), jnp.bfloat16),
            pltpu.SemaphoreType.DMA((NT,)),
            pltpu.SemaphoreType.DMA((NT,)),
        ],
        compiler_params=pltpu.CompilerParams(
            collective_id=0, vmem_limit_bytes=60 * 1024 * 1024
        ),
    )(A2, W)


def kernel(O, Wo):
    B, S, H, D = O.shape
    A = O.reshape(S, H * D).astype(jnp.bfloat16)
    W = Wo.astype(jnp.bfloat16)
    my_x = lax.axis_index("x")
    peer_rows = lax.dynamic_slice(A, ((1 - my_x) * S_HALF, 0), (S_HALF, K))
    own_rows = lax.dynamic_slice(A, (my_x * S_HALF, 0), (S_HALF, K))
    A2 = jnp.stack([peer_rows, own_rows])
    out = _fused(A2, W)
    return out.reshape(B, S_HALF, N)


# device time: 252550 ns/iter; 1.7755x vs baseline; 1.2396x over previous
import jax
import jax.numpy as jnp
from jax import lax
from jax.experimental import pallas as pl
from jax.experimental.pallas import tpu as pltpu

S_HALF = 1024
K = 4096
N = 8192
NH = N // 2
NT2 = 16
TN = NH // NT2
GROUP = 4
NG = NT2 // GROUP
GT = GROUP * TN


def _fused(A3, Wz):

    def body(a_ref, w_ref, o_ref, xsend, xrecv, final, zrecv,
             xsend_sems, xrecv_sems, zsend_sems, zrecv_sems):
        m = pl.program_id(0)
        j = pl.program_id(1)

        my_x = lax.axis_index("x")
        my_y = lax.axis_index("y")
        my_z = lax.axis_index("z")
        xpeer = (1 - my_x, my_y, my_z)
        zpeer = (my_x, my_y, 1 - my_z)

        @pl.when((m == 0) & (j == 0))
        def _():
            barrier = pltpu.get_barrier_semaphore()
            for nbr in (xpeer, zpeer):
                pl.semaphore_signal(
                    barrier, inc=1, device_id=nbr,
                    device_id_type=pl.DeviceIdType.MESH,
                )
            pl.semaphore_wait(barrier, 2)

        def x_rdma(jj):
            return pltpu.make_async_remote_copy(
                src_ref=xsend.at[jj],
                dst_ref=xrecv.at[jj],
                send_sem=xsend_sems.at[jj],
                recv_sem=xrecv_sems.at[jj],
                device_id=xpeer,
                device_id_type=pl.DeviceIdType.MESH,
            )

        def z_rdma(g):
            return pltpu.make_async_remote_copy(
                src_ref=final.at[:, pl.ds(g * GT, GT)],
                dst_ref=zrecv.at[:, pl.ds(g * GT, GT)],
                send_sem=zsend_sems.at[g],
                recv_sem=zrecv_sems.at[g],
                device_id=zpeer,
                device_id_type=pl.DeviceIdType.MESH,
            )

        @pl.when(m == 0)
        def _():
            p = jnp.dot(a_ref[0], w_ref[...],
                        preferred_element_type=jnp.float32)
            xsend[pl.ds(j, 1), :, :] = p.astype(jnp.bfloat16)[None]
            x_rdma(j).start()

        @pl.when(m == 1)
        def _():
            p = jnp.dot(a_ref[0], w_ref[...],
                        preferred_element_type=jnp.float32)
            x_rdma(j).wait()
            r = p + xrecv[pl.ds(j, 1), :, :][0].astype(jnp.float32)
            o_ref[0, :, :] = r
            final[:, pl.ds(j * TN, TN)] = r.astype(jnp.bfloat16)

            @pl.when(j % GROUP == GROUP - 1)
            def _():
                z_rdma(j // GROUP).start()

        @pl.when(m == 2)
        def _():
            @pl.when(j % GROUP == 0)
            def _():
                z_rdma(j // GROUP).wait()

            o_ref[0, :, :] = zrecv[:, pl.ds(j * TN, TN)].astype(jnp.float32)

    return pl.pallas_call(
        body,
        grid=(3, NT2),
        in_specs=[
            pl.BlockSpec((1, S_HALF, K), lambda m, j: (m, 0, 0)),
            pl.BlockSpec((K, TN), lambda m, j: (0, j)),
        ],
        out_specs=pl.BlockSpec(
            (1, S_HALF, TN),
            lambda m, j: (m, 0, j),
        ),
        out_shape=jax.ShapeDtypeStruct((3, S_HALF, NH), jnp.float32),
        scratch_shapes=[
            pltpu.VMEM((NT2, S_HALF, TN), jnp.bfloat16),
            pltpu.VMEM((NT2, S_HALF, TN), jnp.bfloat16),
            pltpu.VMEM((S_HALF, NH), jnp.bfloat16),
            pltpu.VMEM((S_HALF, NH), jnp.bfloat16),
            pltpu.SemaphoreType.DMA((NT2,)),
            pltpu.SemaphoreType.DMA((NT2,)),
            pltpu.SemaphoreType.DMA((NG,)),
            pltpu.SemaphoreType.DMA((NG,)),
        ],
        compiler_params=pltpu.CompilerParams(
            collective_id=0, vmem_limit_bytes=67043328
        ),
    )(A3, Wz)


def kernel(O, Wo):
    B, S, H, D = O.shape
    A = O.reshape(S, H * D).astype(jnp.bfloat16)
    my_x = lax.axis_index("x")
    my_z = lax.axis_index("z")
    peer_rows = lax.dynamic_slice(A, ((1 - my_x) * S_HALF, 0), (S_HALF, K))
    own_rows = lax.dynamic_slice(A, (my_x * S_HALF, 0), (S_HALF, K))
    A3 = jnp.stack([peer_rows, own_rows, own_rows])
    Wz = lax.dynamic_slice(Wo.astype(jnp.bfloat16), (0, my_z * NH), (K, NH))
    out3 = _fused(A3, Wz)
    first = lax.dynamic_index_in_dim(out3, 1 + my_z, axis=0, keepdims=False)
    second = lax.dynamic_index_in_dim(out3, 2 - my_z, axis=0, keepdims=False)
    out = jnp.concatenate([first, second], axis=1)
    return out.reshape(B, S_HALF, N)


# device time: 244145 ns/iter; 1.8367x vs baseline; 1.0344x over previous
import jax
import jax.numpy as jnp
from jax import lax
from jax.experimental import pallas as pl
from jax.experimental.pallas import tpu as pltpu

S_HALF = 1024
K = 4096
N = 8192
NH = N // 2
NT2 = 16
TN = NH // NT2
GROUP = 2
NG = NT2 // GROUP
GT = GROUP * TN


def _fused(A3, Wz):

    def body(a_ref, w_ref, o_ref, xsend, xrecv, final, zrecv,
             xsend_sems, xrecv_sems, zsend_sems, zrecv_sems):
        m = pl.program_id(0)
        j = pl.program_id(1)

        my_x = lax.axis_index("x")
        my_y = lax.axis_index("y")
        my_z = lax.axis_index("z")
        xpeer = (1 - my_x, my_y, my_z)
        zpeer = (my_x, my_y, 1 - my_z)

        @pl.when((m == 0) & (j == 0))
        def _():
            barrier = pltpu.get_barrier_semaphore()
            for nbr in (xpeer, zpeer):
                pl.semaphore_signal(
                    barrier, inc=1, device_id=nbr,
                    device_id_type=pl.DeviceIdType.MESH,
                )
            pl.semaphore_wait(barrier, 2)

        def x_rdma(jj):
            return pltpu.make_async_remote_copy(
                src_ref=xsend.at[jj],
                dst_ref=xrecv.at[jj],
                send_sem=xsend_sems.at[jj],
                recv_sem=xrecv_sems.at[jj],
                device_id=xpeer,
                device_id_type=pl.DeviceIdType.MESH,
            )

        def z_rdma(g):
            return pltpu.make_async_remote_copy(
                src_ref=final.at[:, pl.ds(g * GT, GT)],
                dst_ref=zrecv.at[:, pl.ds(g * GT, GT)],
                send_sem=zsend_sems.at[g],
                recv_sem=zrecv_sems.at[g],
                device_id=zpeer,
                device_id_type=pl.DeviceIdType.MESH,
            )

        @pl.when(m == 0)
        def _():
            p = jnp.dot(a_ref[0], w_ref[...],
                        preferred_element_type=jnp.float32)
            xsend[pl.ds(j, 1), :, :] = p.astype(jnp.bfloat16)[None]
            x_rdma(j).start()

        @pl.when(m == 1)
        def _():
            p = jnp.dot(a_ref[0], w_ref[...],
                        preferred_element_type=jnp.float32)
            x_rdma(j).wait()
            r = p + xrecv[pl.ds(j, 1), :, :][0].astype(jnp.float32)
            o_ref[0, :, :] = r
            final[:, pl.ds(j * TN, TN)] = r.astype(jnp.bfloat16)

            @pl.when(j % GROUP == GROUP - 1)
            def _():
                z_rdma(j // GROUP).start()

        @pl.when(m == 2)
        def _():
            @pl.when(j % GROUP == 0)
            def _():
                z_rdma(j // GROUP).wait()

            o_ref[0, :, :] = zrecv[:, pl.ds(j * TN, TN)].astype(jnp.float32)

    return pl.pallas_call(
        body,
        grid=(3, NT2),
        in_specs=[
            pl.BlockSpec((1, S_HALF, K), lambda m, j: (m, 0, 0)),
            pl.BlockSpec((K, TN), lambda m, j: (0, j)),
        ],
        out_specs=pl.BlockSpec(
            (1, S_HALF, TN),
            lambda m, j: (m, 0, j),
        ),
        out_shape=jax.ShapeDtypeStruct((3, S_HALF, NH), jnp.float32),
        scratch_shapes=[
            pltpu.VMEM((NT2, S_HALF, TN), jnp.bfloat16),
            pltpu.VMEM((NT2, S_HALF, TN), jnp.bfloat16),
            pltpu.VMEM((S_HALF, NH), jnp.bfloat16),
            pltpu.VMEM((S_HALF, NH), jnp.bfloat16),
            pltpu.SemaphoreType.DMA((NT2,)),
            pltpu.SemaphoreType.DMA((NT2,)),
            pltpu.SemaphoreType.DMA((NG,)),
            pltpu.SemaphoreType.DMA((NG,)),
        ],
        compiler_params=pltpu.CompilerParams(
            collective_id=0, vmem_limit_bytes=67043328
        ),
    )(A3, Wz)


def kernel(O, Wo):
    B, S, H, D = O.shape
    A = O.reshape(S, H * D).astype(jnp.bfloat16)
    my_x = lax.axis_index("x")
    my_z = lax.axis_index("z")
    peer_rows = lax.dynamic_slice(A, ((1 - my_x) * S_HALF, 0), (S_HALF, K))
    own_rows = lax.dynamic_slice(A, (my_x * S_HALF, 0), (S_HALF, K))
    A3 = jnp.stack([peer_rows, own_rows, own_rows])
    Wz = lax.dynamic_slice(Wo, (0, my_z * NH), (K, NH)).astype(jnp.bfloat16)
    out3 = _fused(A3, Wz)
    first = lax.dynamic_index_in_dim(out3, 1 + my_z, axis=0, keepdims=False)
    second = lax.dynamic_index_in_dim(out3, 2 - my_z, axis=0, keepdims=False)
    out = jnp.concatenate([first, second], axis=1)
    return out.reshape(B, S_HALF, N)
